# Initial kernel scaffold; baseline (speedup 1.0000x reference)
import jax
import jax.numpy as jnp
from jax.experimental import pallas as pl


def kernel(boxes, scores):
    raise NotImplementedError("write your pallas kernel here")


# speedup vs baseline: 48.8419x; 48.8419x over previous
import jax
import jax.numpy as jnp
from jax import lax
from jax.experimental import pallas as pl
from jax.experimental.pallas import tpu as pltpu

_IOU_T = 0.7
_B = 128


def _nms_body(x1c, y1c, x2c, y2c, x1r, y1r, x2r, y2r, s_ref, out_ref, keep_ref):
    nb = out_ref.shape[0]

    def init_keep(j, c):
        keep_ref[j] = jnp.ones((1, _B), jnp.float32)
        return c

    lax.fori_loop(0, nb, init_keep, 0)

    def iou_gt(rx1, ry1, rx2, ry2, rar, j):
        cx1 = x1c[j]
        cy1 = y1c[j]
        cx2 = x2c[j]
        cy2 = y2c[j]
        car = (cx2 - cx1) * (cy2 - cy1)
        x1 = jnp.maximum(rx1, cx1)
        y1 = jnp.maximum(ry1, cy1)
        x2 = jnp.minimum(rx2, cx2)
        y2 = jnp.minimum(ry2, cy2)
        iw = jnp.maximum(x2 - x1, 0.0)
        ih = jnp.maximum(y2 - y1, 0.0)
        inter = iw * ih
        union = rar + car - inter
        return (inter / jnp.maximum(union, 1e-8)) > _IOU_T

    def body_i(i, carry):
        rx1 = x1r[i]
        ry1 = y1r[i]
        rx2 = x2r[i]
        ry2 = y2r[i]
        rar = (rx2 - rx1) * (ry2 - ry1)

        rid = lax.broadcasted_iota(jnp.int32, (_B, 1), 0)
        cid = lax.broadcasted_iota(jnp.int32, (1, _B), 1)
        diag = iou_gt(rx1, ry1, rx2, ry2, rar, i) & (cid > rid)
        dmat = diag.astype(jnp.bfloat16)
        keep_in = keep_ref[i]

        def jcond(st):
            return st[1]

        def jbody(st):
            k, _ = st
            cnt = jnp.dot(k.astype(jnp.bfloat16), dmat,
                          preferred_element_type=jnp.float32)
            k_new = keep_in * (1.0 - (cnt > 0.5).astype(jnp.float32))
            return k_new, jnp.any(k_new != k)

        kloc, _ = lax.while_loop(jcond, jbody, (keep_in, jnp.array(True)))
        keep_ref[i] = kloc
        kb = kloc.astype(jnp.bfloat16)

        def body_j(j, c):
            cmat = iou_gt(rx1, ry1, rx2, ry2, rar, j).astype(jnp.bfloat16)
            cnt = jnp.dot(kb, cmat, preferred_element_type=jnp.float32)
            keep_ref[j] = keep_ref[j] * (1.0 - (cnt > 0.5).astype(jnp.float32))
            return c

        lax.fori_loop(i + 1, nb, body_j, 0)
        return carry

    lax.fori_loop(0, nb, body_i, 0)

    def finish(j, c):
        out_ref[j] = s_ref[j] * keep_ref[j]
        return c

    lax.fori_loop(0, nb, finish, 0)


def kernel(boxes, scores):
    n = boxes.shape[0]
    nb = pl.cdiv(n, _B)
    w = nb * _B
    order = jnp.argsort(-scores)
    b = jnp.concatenate(
        [boxes[order], jnp.zeros((w - n, 4), boxes.dtype)], axis=0)
    s = jnp.concatenate([scores[order], jnp.zeros((w - n,), scores.dtype)])
    cols = [b[:, k].reshape(nb, 1, _B) for k in range(4)]
    rows = [b[:, k].reshape(nb, _B, 1) for k in range(4)]
    vals = pl.pallas_call(
        _nms_body,
        out_shape=jax.ShapeDtypeStruct((nb, 1, _B), jnp.float32),
        scratch_shapes=[pltpu.VMEM((nb, 1, _B), jnp.float32)],
    )(*cols, *rows, s.reshape(nb, 1, _B))
    return jnp.zeros((n,), scores.dtype).at[order].set(vals.reshape(w)[:n])


# speedup vs baseline: 60.0691x; 1.2299x over previous
import jax
import jax.numpy as jnp
from jax import lax
from jax.experimental import pallas as pl
from jax.experimental.pallas import tpu as pltpu

_IOU_T = 0.7
_B = 128


def _nms_body(x1c, y1c, x2c, y2c, x1p, y1p, x2p, y2p,
              x1r, y1r, x2r, y2r, s_ref, out_ref, keep_ref):
    nb = out_ref.shape[0]
    nbp = nb // 2
    w2 = 2 * _B

    def init_keep(j, c):
        keep_ref[j] = jnp.ones((1, _B), jnp.float32)
        return c

    lax.fori_loop(0, nb, init_keep, 0)

    def iou_gt(rx1, ry1, rx2, ry2, rar, cx1, cy1, cx2, cy2):
        car = (cx2 - cx1) * (cy2 - cy1)
        x1 = jnp.maximum(rx1, cx1)
        y1 = jnp.maximum(ry1, cy1)
        x2 = jnp.minimum(rx2, cx2)
        y2 = jnp.minimum(ry2, cy2)
        iw = jnp.maximum(x2 - x1, 0.0)
        ih = jnp.maximum(y2 - y1, 0.0)
        inter = iw * ih
        union = rar + car - inter
        return (inter / jnp.maximum(union, 1e-8)) > _IOU_T

    def body_i(i, carry):
        rx1 = x1r[i]
        ry1 = y1r[i]
        rx2 = x2r[i]
        ry2 = y2r[i]
        rar = (rx2 - rx1) * (ry2 - ry1)
        rid = i * _B + lax.broadcasted_iota(jnp.int32, (_B, 1), 0)

        cid = i * _B + lax.broadcasted_iota(jnp.int32, (1, _B), 1)
        diag = iou_gt(rx1, ry1, rx2, ry2, rar,
                      x1c[i], y1c[i], x2c[i], y2c[i]) & (cid > rid)
        dmat = diag.astype(jnp.bfloat16)
        keep_in = keep_ref[i]

        def jcond(st):
            return st[1]

        def jbody(st):
            k, _ = st
            cnt = jnp.dot(k.astype(jnp.bfloat16), dmat,
                          preferred_element_type=jnp.float32)
            k_new = keep_in * (1.0 - (cnt > 0.5).astype(jnp.float32))
            return k_new, jnp.any(k_new != k)

        kloc, _ = lax.while_loop(jcond, jbody, (keep_in, jnp.array(True)))
        keep_ref[i] = kloc
        kb = kloc.astype(jnp.bfloat16)

        def body_j(p, c):
            pcid = p * w2 + lax.broadcasted_iota(jnp.int32, (1, w2), 1)
            m = iou_gt(rx1, ry1, rx2, ry2, rar,
                       x1p[p], y1p[p], x2p[p], y2p[p]) & (pcid > rid)
            cnt = jnp.dot(kb, m.astype(jnp.bfloat16),
                          preferred_element_type=jnp.float32)
            supf = 1.0 - (cnt > 0.5).astype(jnp.float32)
            keep_ref[2 * p] = keep_ref[2 * p] * supf[:, :_B]
            keep_ref[2 * p + 1] = keep_ref[2 * p + 1] * supf[:, _B:]
            return c

        lax.fori_loop(i // 2, nbp, body_j, 0)
        return carry

    lax.fori_loop(0, nb, body_i, 0)

    def finish(j, c):
        out_ref[j] = s_ref[j] * keep_ref[j]
        return c

    lax.fori_loop(0, nb, finish, 0)


def kernel(boxes, scores):
    n = boxes.shape[0]
    nb = pl.cdiv(n, _B)
    w = nb * _B
    order = jnp.argsort(-scores)
    b = jnp.concatenate(
        [boxes[order], jnp.zeros((w - n, 4), boxes.dtype)], axis=0)
    s = jnp.concatenate([scores[order], jnp.zeros((w - n,), scores.dtype)])
    cols = [b[:, k].reshape(nb, 1, _B) for k in range(4)]
    pairs = [b[:, k].reshape(nb // 2, 1, 2 * _B) for k in range(4)]
    rows = [b[:, k].reshape(nb, _B, 1) for k in range(4)]
    vals = pl.pallas_call(
        _nms_body,
        out_shape=jax.ShapeDtypeStruct((nb, 1, _B), jnp.float32),
        scratch_shapes=[pltpu.VMEM((nb, 1, _B), jnp.float32)],
    )(*cols, *pairs, *rows, s.reshape(nb, 1, _B))
    return jnp.zeros((n,), scores.dtype).at[order].set(vals.reshape(w)[:n])


# speedup vs baseline: 66.6456x; 1.1095x over previous
import functools

import jax
import jax.numpy as jnp
from jax import lax
from jax.experimental import pallas as pl
from jax.experimental.pallas import tpu as pltpu
from jax.experimental.pallas import tpu_sc as plsc

_IOU_T = 0.7
_B = 128

_NW = 32
_CHUNK = 80
_NCH = 2
_D = 128


def _sc_gather(table, idx2):
    mesh = plsc.VectorSubcoreMesh(core_axis_name="c", subcore_axis_name="s")

    @functools.partial(
        pl.kernel, mesh=mesh,
        out_type=jax.ShapeDtypeStruct((_NW * _NCH, _CHUNK, _D), jnp.float32),
        scratch_types=[
            pltpu.VMEM((_NCH, _CHUNK), jnp.int32),
            pltpu.VMEM((_NCH, _CHUNK, _D), jnp.float32),
            pltpu.SemaphoreType.DMA,
        ],
    )
    def k(table_hbm, idx_hbm, out_hbm, idx_v, rows_v, sem):
        wid = lax.axis_index("s") * 2 + lax.axis_index("c")
        base = wid * _NCH
        pltpu.sync_copy(idx_hbm.at[pl.ds(base, _NCH)], idx_v)
        for j in range(_NCH):
            pltpu.async_copy(table_hbm.at[idx_v.at[j]], rows_v.at[j], sem).wait()
        pltpu.sync_copy(rows_v, out_hbm.at[pl.ds(base, _NCH)])

    return k(table, idx2)


def _sc_scatter(vals3, idx2, w):
    mesh = plsc.VectorSubcoreMesh(core_axis_name="c", subcore_axis_name="s")

    @functools.partial(
        pl.kernel, mesh=mesh,
        out_type=jax.ShapeDtypeStruct((w, _D), jnp.float32),
        scratch_types=[
            pltpu.VMEM((_NCH, _CHUNK), jnp.int32),
            pltpu.VMEM((_NCH, _CHUNK, _D), jnp.float32),
            pltpu.SemaphoreType.DMA,
        ],
    )
    def k(vals_hbm, idx_hbm, out_hbm, idx_v, rows_v, sem):
        wid = lax.axis_index("s") * 2 + lax.axis_index("c")
        base = wid * _NCH
        pltpu.sync_copy(idx_hbm.at[pl.ds(base, _NCH)], idx_v)
        pltpu.sync_copy(vals_hbm.at[pl.ds(base, _NCH)], rows_v)
        for j in range(_NCH):
            pltpu.async_copy(rows_v.at[j], out_hbm.at[idx_v.at[j]], sem).wait()

    return k(vals3, idx2)


def _nms_body(x1c, y1c, x2c, y2c, x1p, y1p, x2p, y2p,
              x1r, y1r, x2r, y2r, s_ref, out_ref, keep_ref, car_ref):
    nb = out_ref.shape[0]
    nbp = nb // 2
    w2 = 2 * _B

    def init_keep(j, c):
        keep_ref[j] = jnp.ones((1, _B), jnp.float32)
        car_ref[j] = (x2c[j] - x1c[j]) * (y2c[j] - y1c[j])
        return c

    lax.fori_loop(0, nb, init_keep, 0)

    def iou_gt(rx1, ry1, rx2, ry2, rar, cx1, cy1, cx2, cy2, car):
        x1 = jnp.maximum(rx1, cx1)
        y1 = jnp.maximum(ry1, cy1)
        x2 = jnp.minimum(rx2, cx2)
        y2 = jnp.minimum(ry2, cy2)
        iw = jnp.maximum(x2 - x1, 0.0)
        ih = jnp.maximum(y2 - y1, 0.0)
        inter = iw * ih
        union = rar + car - inter
        return (inter / jnp.maximum(union, 1e-8)) > _IOU_T

    def body_i(i, carry):
        rx1 = x1r[i]
        ry1 = y1r[i]
        rx2 = x2r[i]
        ry2 = y2r[i]
        rar = (rx2 - rx1) * (ry2 - ry1)
        rid = i * _B + lax.broadcasted_iota(jnp.int32, (_B, 1), 0)

        cid = i * _B + lax.broadcasted_iota(jnp.int32, (1, _B), 1)
        diag = iou_gt(rx1, ry1, rx2, ry2, rar,
                      x1c[i], y1c[i], x2c[i], y2c[i], car_ref[i]) & (cid > rid)
        dmat = diag.astype(jnp.bfloat16)
        keep_in = keep_ref[i]

        def jcond(st):
            return st[1]

        def jbody(st):
            k, _ = st
            cnt = jnp.dot(k.astype(jnp.bfloat16), dmat,
                          preferred_element_type=jnp.float32)
            k_new = keep_in * (cnt <= 0.5).astype(jnp.float32)
            return k_new, jnp.any(k_new != k)

        kloc, _ = lax.while_loop(jcond, jbody, (keep_in, jnp.array(True)))
        keep_ref[i] = kloc
        kb = kloc.astype(jnp.bfloat16)

        def apply_pair(p, masked):
            car = jnp.concatenate([car_ref[2 * p], car_ref[2 * p + 1]], axis=1)
            m = iou_gt(rx1, ry1, rx2, ry2, rar,
                       x1p[p], y1p[p], x2p[p], y2p[p], car)
            if masked:
                pcid = p * w2 + lax.broadcasted_iota(jnp.int32, (1, w2), 1)
                m = m & (pcid > rid)
            cnt = jnp.dot(kb, m.astype(jnp.bfloat16),
                          preferred_element_type=jnp.float32)
            keepf = (cnt <= 0.5).astype(jnp.float32)
            keep_ref[2 * p] = keep_ref[2 * p] * keepf[:, :_B]
            keep_ref[2 * p + 1] = keep_ref[2 * p + 1] * keepf[:, _B:]

        apply_pair(i // 2, True)
        start = i // 2 + 1
        half = (nbp - start) // 2

        def body_j2(q, c):
            p = start + 2 * q
            apply_pair(p, False)
            apply_pair(p + 1, False)
            return c

        lax.fori_loop(0, half, body_j2, 0)

        @pl.when((nbp - start) % 2 == 1)
        def _tail():
            apply_pair(start + 2 * half, False)

        return carry

    lax.fori_loop(0, nb, body_i, 0)

    def finish(j, c):
        out_ref[j] = s_ref[j] * keep_ref[j]
        return c

    lax.fori_loop(0, nb, finish, 0)


def kernel(boxes, scores):
    n = boxes.shape[0]
    nb = pl.cdiv(n, _B)
    w = nb * _B
    order = jnp.argsort(-scores).astype(jnp.int32)
    idx2 = jnp.concatenate(
        [order, jnp.arange(n, w, dtype=jnp.int32)]).reshape(_NW * _NCH, _CHUNK)
    table = (jnp.zeros((w, _D), jnp.float32)
             .at[:n, :4].set(boxes).at[:n, 4].set(scores))
    g = _sc_gather(table, idx2).reshape(w, _D)
    b = g[:, :4]
    s = g[:, 4]
    cols = [b[:, k].reshape(nb, 1, _B) for k in range(4)]
    pairs = [b[:, k].reshape(nb // 2, 1, 2 * _B) for k in range(4)]
    rows = [b[:, k].reshape(nb, _B, 1) for k in range(4)]
    vals = pl.pallas_call(
        _nms_body,
        out_shape=jax.ShapeDtypeStruct((nb, 1, _B), jnp.float32),
        scratch_shapes=[pltpu.VMEM((nb, 1, _B), jnp.float32),
                        pltpu.VMEM((nb, 1, _B), jnp.float32)],
    )(*cols, *pairs, *rows, s.reshape(nb, 1, _B))
    vals3 = (jnp.zeros((w, _D), jnp.float32)
             .at[:, 0].set(vals.reshape(w)).reshape(_NW * _NCH, _CHUNK, _D))
    return _sc_scatter(vals3, idx2, w)[:n, 0]


# speedup vs baseline: 67.9141x; 1.0190x over previous
import functools

import jax
import jax.numpy as jnp
from jax import lax
from jax.experimental import pallas as pl
from jax.experimental.pallas import tpu as pltpu
from jax.experimental.pallas import tpu_sc as plsc

_IOU_T = 0.7
_B = 128

_NW = 32
_CHUNK = 80
_NCH = 2
_D = 128


def _sc_gather(table, idx2):
    mesh = plsc.VectorSubcoreMesh(core_axis_name="c", subcore_axis_name="s")

    @functools.partial(
        pl.kernel, mesh=mesh,
        out_type=jax.ShapeDtypeStruct((_NW * _NCH, _CHUNK, _D), jnp.float32),
        scratch_types=[
            pltpu.VMEM((_NCH, _CHUNK), jnp.int32),
            pltpu.VMEM((_NCH, _CHUNK, _D), jnp.float32),
            pltpu.SemaphoreType.DMA,
        ],
    )
    def k(table_hbm, idx_hbm, out_hbm, idx_v, rows_v, sem):
        wid = lax.axis_index("s") * 2 + lax.axis_index("c")
        base = wid * _NCH
        pltpu.sync_copy(idx_hbm.at[pl.ds(base, _NCH)], idx_v)
        for j in range(_NCH):
            pltpu.async_copy(table_hbm.at[idx_v.at[j]], rows_v.at[j], sem).wait()
        pltpu.sync_copy(rows_v, out_hbm.at[pl.ds(base, _NCH)])

    return k(table, idx2)


def _sc_scatter(vals3, idx2, w):
    mesh = plsc.VectorSubcoreMesh(core_axis_name="c", subcore_axis_name="s")

    @functools.partial(
        pl.kernel, mesh=mesh,
        out_type=jax.ShapeDtypeStruct((w, _D), jnp.float32),
        scratch_types=[
            pltpu.VMEM((_NCH, _CHUNK), jnp.int32),
            pltpu.VMEM((_NCH, _CHUNK, _D), jnp.float32),
            pltpu.SemaphoreType.DMA,
        ],
    )
    def k(vals_hbm, idx_hbm, out_hbm, idx_v, rows_v, sem):
        wid = lax.axis_index("s") * 2 + lax.axis_index("c")
        base = wid * _NCH
        pltpu.sync_copy(idx_hbm.at[pl.ds(base, _NCH)], idx_v)
        pltpu.sync_copy(vals_hbm.at[pl.ds(base, _NCH)], rows_v)
        for j in range(_NCH):
            pltpu.async_copy(rows_v.at[j], out_hbm.at[idx_v.at[j]], sem).wait()

    return k(vals3, idx2)


def _nms_body(x1c, y1c, x2c, y2c, x1p, y1p, x2p, y2p,
              x1r, y1r, x2r, y2r, s_ref, out_ref, keep_ref, car_ref):
    nb = out_ref.shape[0]
    nbp = nb // 2
    w2 = 2 * _B

    def init_keep(j, c):
        keep_ref[j] = jnp.ones((1, _B), jnp.float32)
        car_ref[j] = (x2c[j] - x1c[j]) * (y2c[j] - y1c[j])
        return c

    lax.fori_loop(0, nb, init_keep, 0)

    def iou_gt(rx1, ry1, rx2, ry2, rar, cx1, cy1, cx2, cy2, car):
        x1 = jnp.maximum(rx1, cx1)
        y1 = jnp.maximum(ry1, cy1)
        x2 = jnp.minimum(rx2, cx2)
        y2 = jnp.minimum(ry2, cy2)
        iw = jnp.maximum(x2 - x1, 0.0)
        ih = jnp.maximum(y2 - y1, 0.0)
        inter = iw * ih
        union = rar + car - inter
        return (inter / jnp.maximum(union, 1e-8)) > _IOU_T

    def body_i(i, carry):
        rx1 = x1r[i]
        ry1 = y1r[i]
        rx2 = x2r[i]
        ry2 = y2r[i]
        rar = (rx2 - rx1) * (ry2 - ry1)
        rid = i * _B + lax.broadcasted_iota(jnp.int32, (_B, 1), 0)

        cid = i * _B + lax.broadcasted_iota(jnp.int32, (1, _B), 1)
        diag = iou_gt(rx1, ry1, rx2, ry2, rar,
                      x1c[i], y1c[i], x2c[i], y2c[i], car_ref[i]) & (cid > rid)
        dmat = diag.astype(jnp.bfloat16)
        keep_in = keep_ref[i]

        def jcond(st):
            return st[1]

        def jbody(st):
            k, _ = st
            cnt = jnp.dot(k.astype(jnp.bfloat16), dmat,
                          preferred_element_type=jnp.float32)
            k_new = keep_in * (cnt <= 0.5).astype(jnp.float32)
            return k_new, jnp.any(k_new != k)

        kloc, _ = lax.while_loop(jcond, jbody, (keep_in, jnp.array(True)))
        keep_ref[i] = kloc
        kb = kloc.astype(jnp.bfloat16)

        def apply_pair(p, masked):
            car = jnp.concatenate([car_ref[2 * p], car_ref[2 * p + 1]], axis=1)
            m = iou_gt(rx1, ry1, rx2, ry2, rar,
                       x1p[p], y1p[p], x2p[p], y2p[p], car)
            if masked:
                pcid = p * w2 + lax.broadcasted_iota(jnp.int32, (1, w2), 1)
                m = m & (pcid > rid)
            cnt = jnp.dot(kb, m.astype(jnp.bfloat16),
                          preferred_element_type=jnp.float32)
            keepf = (cnt <= 0.5).astype(jnp.float32)
            keep_ref[2 * p] = keep_ref[2 * p] * keepf[:, :_B]
            keep_ref[2 * p + 1] = keep_ref[2 * p + 1] * keepf[:, _B:]

        apply_pair(i // 2, True)
        start = i // 2 + 1
        cnt = nbp - start
        quads = cnt // 4

        def body_j4(q, c):
            p = start + 4 * q
            apply_pair(p, False)
            apply_pair(p + 1, False)
            apply_pair(p + 2, False)
            apply_pair(p + 3, False)
            return c

        lax.fori_loop(0, quads, body_j4, 0)
        rest = start + 4 * quads

        @pl.when(cnt % 4 >= 1)
        def _tail1():
            apply_pair(rest, False)

        @pl.when(cnt % 4 >= 2)
        def _tail2():
            apply_pair(rest + 1, False)

        @pl.when(cnt % 4 >= 3)
        def _tail3():
            apply_pair(rest + 2, False)

        return carry

    lax.fori_loop(0, nb, body_i, 0)

    def finish(j, c):
        out_ref[j] = s_ref[j] * keep_ref[j]
        return c

    lax.fori_loop(0, nb, finish, 0)


def kernel(boxes, scores):
    n = boxes.shape[0]
    nb = pl.cdiv(n, _B)
    w = nb * _B
    order = jnp.argsort(-scores).astype(jnp.int32)
    idx2 = jnp.concatenate(
        [order, jnp.arange(n, w, dtype=jnp.int32)]).reshape(_NW * _NCH, _CHUNK)
    table = (jnp.zeros((w, _D), jnp.float32)
             .at[:n, :4].set(boxes).at[:n, 4].set(scores))
    g = _sc_gather(table, idx2).reshape(w, _D)
    b = g[:, :4]
    s = g[:, 4]
    cols = [b[:, k].reshape(nb, 1, _B) for k in range(4)]
    pairs = [b[:, k].reshape(nb // 2, 1, 2 * _B) for k in range(4)]
    rows = [b[:, k].reshape(nb, _B, 1) for k in range(4)]
    vals = pl.pallas_call(
        _nms_body,
        out_shape=jax.ShapeDtypeStruct((nb, 1, _B), jnp.float32),
        scratch_shapes=[pltpu.VMEM((nb, 1, _B), jnp.float32),
                        pltpu.VMEM((nb, 1, _B), jnp.float32)],
    )(*cols, *pairs, *rows, s.reshape(nb, 1, _B))
    vals3 = (jnp.zeros((w, _D), jnp.float32)
             .at[:, 0].set(vals.reshape(w)).reshape(_NW * _NCH, _CHUNK, _D))
    return _sc_scatter(vals3, idx2, w)[:n, 0]
